# baseline (speedup 1.0000x reference)
import functools

import jax
import jax.numpy as jnp
from jax import lax
from jax.experimental import pallas as pl
from jax.experimental.pallas import tpu as pltpu
from jax.experimental.pallas import tpu_sc as plsc

CH = 128


@functools.lru_cache(maxsize=None)
def _make_gather(V, D, N):
    info = plsc.get_sparse_core_info()
    NC, NS = info.num_cores, info.num_subcores
    NW = NC * NS
    assert N % (NW * CH) == 0
    n_per_w = N // NW
    n_chunks = n_per_w // CH
    n_half = n_chunks // 2
    assert n_chunks % 2 == 0

    mesh = plsc.VectorSubcoreMesh(core_axis_name="c", subcore_axis_name="s")

    @functools.partial(
        pl.kernel,
        mesh=mesh,
        compiler_params=pltpu.CompilerParams(use_tc_tiling_on_sc=False),
        out_type=jax.ShapeDtypeStruct((N, D), jnp.float32),
        scratch_types=[
            pltpu.VMEM((n_chunks, CH), jnp.int32),
            pltpu.VMEM((2, CH, D), jnp.float32),
            pltpu.SemaphoreType.DMA,
            pltpu.SemaphoreType.DMA,
        ],
    )
    def k(x_hbm, table_hbm, out_hbm, idx_v, rows_v, sem0, sem1):
        wid = lax.axis_index("s") * NC + lax.axis_index("c")
        base = wid * n_per_w
        pltpu.sync_copy(x_hbm.at[wid], idx_v)
        pltpu.async_copy(table_hbm.at[idx_v.at[0]], rows_v.at[0], sem0)

        def body(jj, _):
            c0 = 2 * jj
            pltpu.async_copy(table_hbm.at[idx_v.at[c0 + 1]], rows_v.at[1], sem1)
            pltpu.make_async_copy(
                table_hbm.at[idx_v.at[c0]], rows_v.at[0], sem0
            ).wait()
            pltpu.sync_copy(rows_v.at[0], out_hbm.at[pl.ds(base + c0 * CH, CH)])

            @pl.when(jj + 1 < n_half)
            def _():
                pltpu.async_copy(
                    table_hbm.at[idx_v.at[c0 + 2]], rows_v.at[0], sem0
                )

            pltpu.make_async_copy(
                table_hbm.at[idx_v.at[c0 + 1]], rows_v.at[1], sem1
            ).wait()
            pltpu.sync_copy(
                rows_v.at[1], out_hbm.at[pl.ds(base + (c0 + 1) * CH, CH)]
            )
            return 0

        lax.fori_loop(0, n_half, body, 0, unroll=False)

    return k


def kernel(x, table):
    B, L = x.shape
    V, D = table.shape
    N = B * L
    info = plsc.get_sparse_core_info()
    NW = info.num_cores * info.num_subcores
    x_flat = x.reshape(NW, N // (NW * CH), CH).astype(jnp.int32)
    out = _make_gather(V, D, N)(x_flat, table)
    return out.reshape(B, L, D)


# speedup vs baseline: 1.0206x; 1.0206x over previous
import functools

import jax
import jax.numpy as jnp
from jax import lax
from jax.experimental import pallas as pl
from jax.experimental.pallas import tpu as pltpu
from jax.experimental.pallas import tpu_sc as plsc

CH = 128


@functools.lru_cache(maxsize=None)
def _make_gather(V, D, N):
    info = plsc.get_sparse_core_info()
    NC, NS = info.num_cores, info.num_subcores
    NW = NC * NS
    assert N % (NW * CH) == 0
    n_per_w = N // NW
    n_chunks = n_per_w // CH
    n_half = n_chunks // 2
    assert n_chunks % 2 == 0

    mesh = plsc.VectorSubcoreMesh(core_axis_name="c", subcore_axis_name="s")

    NBUF = 8
    assert n_chunks % NBUF == 0
    n_groups = n_chunks // NBUF

    @functools.partial(
        pl.kernel,
        mesh=mesh,
        compiler_params=pltpu.CompilerParams(use_tc_tiling_on_sc=False),
        out_type=jax.ShapeDtypeStruct((N, D), jnp.float32),
        scratch_types=[
            pltpu.VMEM((n_chunks, CH), jnp.int32),
            pltpu.VMEM((NBUF, CH, D), jnp.float32),
            [pltpu.SemaphoreType.DMA] * NBUF,
            [pltpu.SemaphoreType.DMA] * NBUF,
        ],
    )
    def k(x_hbm, table_hbm, out_hbm, idx_v, rows_v, gsems, osems):
        wid = lax.axis_index("s") * NC + lax.axis_index("c")
        base = wid * n_per_w
        pltpu.sync_copy(x_hbm.at[wid], idx_v)
        for b in range(NBUF):
            pltpu.async_copy(table_hbm.at[idx_v.at[b]], rows_v.at[b], gsems[b])

        def body(g, _):
            j0 = g * NBUF
            for b in range(NBUF):
                pltpu.make_async_copy(
                    table_hbm.at[idx_v.at[j0 + b]], rows_v.at[b], gsems[b]
                ).wait()
                pltpu.async_copy(
                    rows_v.at[b],
                    out_hbm.at[pl.ds(base + (j0 + b) * CH, CH)],
                    osems[b],
                )
            for b in range(NBUF):
                pltpu.make_async_copy(
                    rows_v.at[b],
                    out_hbm.at[pl.ds(base + (j0 + b) * CH, CH)],
                    osems[b],
                ).wait()

                @pl.when(g + 1 < n_groups)
                def _():
                    pltpu.async_copy(
                        table_hbm.at[idx_v.at[j0 + NBUF + b]],
                        rows_v.at[b],
                        gsems[b],
                    )

            return 0

        lax.fori_loop(0, n_groups, body, 0, unroll=False)

    return k


def kernel(x, table):
    B, L = x.shape
    V, D = table.shape
    N = B * L
    info = plsc.get_sparse_core_info()
    NW = info.num_cores * info.num_subcores
    x_flat = x.reshape(NW, N // (NW * CH), CH).astype(jnp.int32)
    out = _make_gather(V, D, N)(x_flat, table)
    return out.reshape(B, L, D)
